# baseline (device time: 37774 ns/iter reference)
import jax
import jax.numpy as jnp
from jax import lax
from jax.experimental import pallas as pl
from jax.experimental.pallas import tpu as pltpu

ROWS = 256
K = 512
HALF = 4096
NC = 8
CHUNK = HALF // NC


def kernel(x, W):
    def body(
        x_hbm,
        w_hbm,
        dummy_hbm,
        out_hbm,
        x_vmem,
        w_vmem,
        send_buf,
        recv_buf,
        x_sem,
        w_sems,
        out_sems,
        send_sems,
        recv_sems,
    ):
        my_x = lax.axis_index("x")
        my_y = lax.axis_index("y")
        my_z = lax.axis_index("z")
        partner = (my_x, 1 - my_y, my_z)

        x_copy = pltpu.make_async_copy(x_hbm, x_vmem, x_sem)
        x_copy.start()

        def w_copy(k):
            return pltpu.make_async_copy(
                w_hbm.at[:, pl.ds(k * CHUNK, CHUNK)],
                w_vmem.at[k % 2],
                w_sems.at[k % 2],
            )

        w_copy(0).start()

        barrier_sem = pltpu.get_barrier_semaphore()
        pl.semaphore_signal(
            barrier_sem, inc=1, device_id=partner,
            device_id_type=pl.DeviceIdType.MESH,
        )
        pl.semaphore_wait(barrier_sem, 1)

        x_copy.wait()
        xl = x_vmem[...].astype(jnp.bfloat16)

        def chunk_rdma(k):
            cs = pl.ds(k * CHUNK, CHUNK)
            return pltpu.make_async_remote_copy(
                src_ref=send_buf.at[:, cs],
                dst_ref=recv_buf.at[:, cs],
                send_sem=send_sems.at[k],
                recv_sem=recv_sems.at[k],
                device_id=partner,
                device_id_type=pl.DeviceIdType.MESH,
            )

        s_loc = jnp.zeros((ROWS, 1), jnp.float32)
        for k in range(NC):
            w_copy(k).wait()
            if k + 1 < NC:
                w_copy(k + 1).start()
            wk = w_vmem[k % 2].astype(jnp.bfloat16)
            ek = jnp.exp(jnp.dot(xl, wk, preferred_element_type=jnp.float32))
            cs = pl.ds(k * CHUNK, CHUNK)
            send_buf[:, cs] = ek.astype(jnp.bfloat16)
            chunk_rdma(k).start()
            s_loc = s_loc + jnp.sum(ek, axis=1, keepdims=True)

        s_rem = jnp.zeros((ROWS, 1), jnp.float32)
        for k in range(NC):
            chunk_rdma(k).wait_recv()
            ck = recv_buf[:, pl.ds(k * CHUNK, CHUNK)].astype(jnp.float32)
            s_rem = s_rem + jnp.sum(ck, axis=1, keepdims=True)

        for k in range(NC):
            chunk_rdma(k).wait_send()

        inv = 1.0 / (s_loc + s_rem)
        loc_off = my_y * HALF
        rem_off = (1 - my_y) * HALF

        send_buf[...] = (send_buf[...].astype(jnp.float32) * inv).astype(
            jnp.bfloat16
        )
        out_loc = pltpu.make_async_copy(
            send_buf, out_hbm.at[:, pl.ds(loc_off, HALF)], out_sems.at[0]
        )
        out_loc.start()

        recv_buf[...] = (recv_buf[...].astype(jnp.float32) * inv).astype(
            jnp.bfloat16
        )
        out_rem = pltpu.make_async_copy(
            recv_buf, out_hbm.at[:, pl.ds(rem_off, HALF)], out_sems.at[1]
        )
        out_rem.start()

        out_loc.wait()
        out_rem.wait()

    return pl.pallas_call(
        body,
        out_shape=jax.ShapeDtypeStruct((ROWS, 2 * HALF), jnp.bfloat16),
        in_specs=[
            pl.BlockSpec(memory_space=pltpu.MemorySpace.HBM),
            pl.BlockSpec(memory_space=pltpu.MemorySpace.HBM),
            pl.BlockSpec(memory_space=pltpu.MemorySpace.HBM),
        ],
        out_specs=pl.BlockSpec(memory_space=pltpu.MemorySpace.HBM),
        input_output_aliases={2: 0},
        scratch_shapes=[
            pltpu.VMEM((ROWS, K), jnp.float32),
            pltpu.VMEM((2, K, CHUNK), jnp.float32),
            pltpu.VMEM((ROWS, HALF), jnp.bfloat16),
            pltpu.VMEM((ROWS, HALF), jnp.bfloat16),
            pltpu.SemaphoreType.DMA,
            pltpu.SemaphoreType.DMA((2,)),
            pltpu.SemaphoreType.DMA((2,)),
            pltpu.SemaphoreType.DMA((NC,)),
            pltpu.SemaphoreType.DMA((NC,)),
        ],
        compiler_params=pltpu.CompilerParams(collective_id=0),
    )(
        pltpu.with_memory_space_constraint(x, pltpu.MemorySpace.HBM),
        pltpu.with_memory_space_constraint(W, pltpu.MemorySpace.HBM),
        pltpu.with_memory_space_constraint(
            jnp.zeros((ROWS, 2 * HALF), jnp.bfloat16), pltpu.MemorySpace.HBM
        ),
    )


# device time: 31437 ns/iter; 1.2016x vs baseline; 1.2016x over previous
import jax
import jax.numpy as jnp
from jax import lax
from jax.experimental import pallas as pl
from jax.experimental.pallas import tpu as pltpu

ROWS = 256
K = 512
HALF = 4096
BOUNDS = [0, 256, 512, 1024, 1536, 2048, 2560, 3328, 4096]
NC = len(BOUNDS) - 1
WMAX = max(b - a for a, b in zip(BOUNDS[:-1], BOUNDS[1:]))
NT = 4
TCHUNK = HALF // NT


def kernel(x, W):
    def body(
        x_hbm,
        w_hbm,
        out_hbm,
        x_vmem,
        w_vmem,
        send_buf,
        recv_buf,
        x_sem,
        w_sems,
        out_sems,
        send_sems,
        recv_sems,
    ):
        my_x = lax.axis_index("x")
        my_y = lax.axis_index("y")
        my_z = lax.axis_index("z")
        partner = (my_x, 1 - my_y, my_z)

        x_copy = pltpu.make_async_copy(x_hbm, x_vmem, x_sem)
        x_copy.start()

        def w_copy(k):
            lo, hi = BOUNDS[k], BOUNDS[k + 1]
            return pltpu.make_async_copy(
                w_hbm.at[:, pl.ds(lo, hi - lo)],
                w_vmem.at[k % 2, :, : hi - lo],
                w_sems.at[k % 2],
            )

        w_copy(0).start()

        barrier_sem = pltpu.get_barrier_semaphore()
        pl.semaphore_signal(
            barrier_sem, inc=1, device_id=partner,
            device_id_type=pl.DeviceIdType.MESH,
        )
        pl.semaphore_wait(barrier_sem, 1)

        x_copy.wait()
        xl = x_vmem[...].astype(jnp.bfloat16)

        def chunk_rdma(k):
            cs = pl.ds(BOUNDS[k], BOUNDS[k + 1] - BOUNDS[k])
            return pltpu.make_async_remote_copy(
                src_ref=send_buf.at[:, cs],
                dst_ref=recv_buf.at[:, cs],
                send_sem=send_sems.at[k],
                recv_sem=recv_sems.at[k],
                device_id=partner,
                device_id_type=pl.DeviceIdType.MESH,
            )

        s_loc = jnp.zeros((ROWS, 1), jnp.float32)
        for k in range(NC):
            lo, hi = BOUNDS[k], BOUNDS[k + 1]
            w_copy(k).wait()
            if k + 1 < NC:
                w_copy(k + 1).start()
            wk = w_vmem[k % 2, :, : hi - lo].astype(jnp.bfloat16)
            ek = jnp.exp(jnp.dot(xl, wk, preferred_element_type=jnp.float32))
            send_buf[:, pl.ds(lo, hi - lo)] = ek.astype(jnp.bfloat16)
            chunk_rdma(k).start()
            s_loc = s_loc + jnp.sum(ek, axis=1, keepdims=True)

        s_rem = jnp.zeros((ROWS, 1), jnp.float32)
        for k in range(NC):
            chunk_rdma(k).wait_recv()
            lo, hi = BOUNDS[k], BOUNDS[k + 1]
            ck = recv_buf[:, pl.ds(lo, hi - lo)].astype(jnp.float32)
            s_rem = s_rem + jnp.sum(ck, axis=1, keepdims=True)

        for k in range(NC):
            chunk_rdma(k).wait_send()

        inv = 1.0 / (s_loc + s_rem)
        loc_off = my_y * HALF
        rem_off = (1 - my_y) * HALF

        copies = []
        for j in range(NT):
            cs = pl.ds(j * TCHUNK, TCHUNK)
            send_buf[:, cs] = (
                send_buf[:, cs].astype(jnp.float32) * inv
            ).astype(jnp.bfloat16)
            c = pltpu.make_async_copy(
                send_buf.at[:, cs],
                out_hbm.at[:, pl.ds(loc_off + j * TCHUNK, TCHUNK)],
                out_sems.at[j],
            )
            c.start()
            copies.append(c)
        for j in range(NT):
            cs = pl.ds(j * TCHUNK, TCHUNK)
            recv_buf[:, cs] = (
                recv_buf[:, cs].astype(jnp.float32) * inv
            ).astype(jnp.bfloat16)
            c = pltpu.make_async_copy(
                recv_buf.at[:, cs],
                out_hbm.at[:, pl.ds(rem_off + j * TCHUNK, TCHUNK)],
                out_sems.at[NT + j],
            )
            c.start()
            copies.append(c)
        for c in copies:
            c.wait()

    return pl.pallas_call(
        body,
        out_shape=jax.ShapeDtypeStruct((ROWS, 2 * HALF), jnp.bfloat16),
        in_specs=[
            pl.BlockSpec(memory_space=pltpu.MemorySpace.HBM),
            pl.BlockSpec(memory_space=pltpu.MemorySpace.HBM),
        ],
        out_specs=pl.BlockSpec(memory_space=pltpu.MemorySpace.HBM),
        scratch_shapes=[
            pltpu.VMEM((ROWS, K), jnp.float32),
            pltpu.VMEM((2, K, WMAX), jnp.float32),
            pltpu.VMEM((ROWS, HALF), jnp.bfloat16),
            pltpu.VMEM((ROWS, HALF), jnp.bfloat16),
            pltpu.SemaphoreType.DMA,
            pltpu.SemaphoreType.DMA((2,)),
            pltpu.SemaphoreType.DMA((2 * NT,)),
            pltpu.SemaphoreType.DMA((NC,)),
            pltpu.SemaphoreType.DMA((NC,)),
        ],
        compiler_params=pltpu.CompilerParams(collective_id=0),
    )(
        pltpu.with_memory_space_constraint(x, pltpu.MemorySpace.HBM),
        pltpu.with_memory_space_constraint(W, pltpu.MemorySpace.HBM),
    )


# device time: 26219 ns/iter; 1.4407x vs baseline; 1.1990x over previous
import jax
import jax.numpy as jnp
from jax import lax
from jax.experimental import pallas as pl
from jax.experimental.pallas import tpu as pltpu

ROWS = 256
K = 512
HALF = 4096
Q = HALF // 4
YC = 4
YCW = Q // YC
NREST = 3
NT = 4
TCHUNK = HALF // NT


def kernel(x, W):
    def body(
        x_hbm,
        w_hbm,
        out_hbm,
        x_vmem,
        w_vmem,
        send_buf,
        recv_buf,
        x_sem,
        w_sems,
        out_sems,
        y_send, y_recv,
        xf_send, xf_recv,
        z_send, z_recv,
    ):
        my_x = lax.axis_index("x")
        my_y = lax.axis_index("y")
        my_z = lax.axis_index("z")
        p_y = (my_x, 1 - my_y, my_z)
        p_x = (1 - my_x, my_y, my_z)
        p_z = (my_x, my_y, 1 - my_z)

        qoff = (2 * my_z + my_x) * Q
        qxoff = (2 * my_z + (1 - my_x)) * Q
        zoff = 2 * my_z * Q

        x_copy = pltpu.make_async_copy(x_hbm, x_vmem, x_sem)
        x_copy.start()

        def w_copy(i, col, width):
            return pltpu.make_async_copy(
                w_hbm.at[:, pl.ds(col, width)],
                w_vmem.at[i % 2, :, :width],
                w_sems.at[i % 2],
            )

        sched = [(qoff + j * YCW, YCW) for j in range(YC)]
        sched += [
            (lax.rem(qoff + Q + r * Q, HALF), Q) for r in range(NREST)
        ]

        w_copy(0, *sched[0]).start()

        barrier_sem = pltpu.get_barrier_semaphore()
        for nbr in (p_y, p_x, p_z):
            pl.semaphore_signal(
                barrier_sem, inc=1, device_id=nbr,
                device_id_type=pl.DeviceIdType.MESH,
            )
        pl.semaphore_wait(barrier_sem, 3)

        x_copy.wait()
        xl = x_vmem[...].astype(jnp.bfloat16)

        def y_rdma(j):
            cs = pl.ds(qoff + j * YCW, YCW)
            return pltpu.make_async_remote_copy(
                src_ref=send_buf.at[:, cs],
                dst_ref=recv_buf.at[:, cs],
                send_sem=y_send.at[j],
                recv_sem=y_recv.at[j],
                device_id=p_y,
                device_id_type=pl.DeviceIdType.MESH,
            )

        def x_rdma(j):
            cs = pl.ds(qoff + j * YCW, YCW)
            return pltpu.make_async_remote_copy(
                src_ref=recv_buf.at[:, cs],
                dst_ref=recv_buf.at[:, cs],
                send_sem=xf_send.at[j],
                recv_sem=xf_recv.at[j],
                device_id=p_x,
                device_id_type=pl.DeviceIdType.MESH,
            )

        def x_rdma_recv(j):
            cs = pl.ds(qxoff + j * YCW, YCW)
            return pltpu.make_async_remote_copy(
                src_ref=recv_buf.at[:, cs],
                dst_ref=recv_buf.at[:, cs],
                send_sem=xf_send.at[j],
                recv_sem=xf_recv.at[j],
                device_id=p_x,
                device_id_type=pl.DeviceIdType.MESH,
            )

        def z_rdma(k, src_off):
            cs = pl.ds(src_off, YCW)
            return pltpu.make_async_remote_copy(
                src_ref=recv_buf.at[:, cs],
                dst_ref=recv_buf.at[:, cs],
                send_sem=z_send.at[k],
                recv_sem=z_recv.at[k],
                device_id=p_z,
                device_id_type=pl.DeviceIdType.MESH,
            )

        s_loc = jnp.zeros((ROWS, 1), jnp.float32)
        for i in range(len(sched)):
            col, width = sched[i]
            wc = pltpu.make_async_copy(
                w_hbm.at[:, pl.ds(col, width)],
                w_vmem.at[i % 2, :, :width],
                w_sems.at[i % 2],
            )
            wc.wait()
            if i + 1 < len(sched):
                w_copy(i + 1, *sched[i + 1]).start()
            wk = w_vmem[i % 2, :, :width].astype(jnp.bfloat16)
            ek = jnp.exp(jnp.dot(xl, wk, preferred_element_type=jnp.float32))
            send_buf[:, pl.ds(col, width)] = ek.astype(jnp.bfloat16)
            if i < YC:
                y_rdma(i).start()
            s_loc = s_loc + jnp.sum(ek, axis=1, keepdims=True)

            if YC <= i < YC + 3:
                j = i - YC
                y_rdma(j).wait_recv()
                x_rdma(j).start()
                z_rdma(j, qoff + j * YCW).start()

        y_rdma(3).wait_recv()
        x_rdma(3).start()
        z_rdma(3, qoff + 3 * YCW).start()

        for j in range(YC):
            x_rdma_recv(j).wait_recv()
            z_rdma(YC + j, qxoff + j * YCW).start()

        for k in range(2 * YC):
            z_rdma(k, zoff).wait_recv()

        for j in range(YC):
            y_rdma(j).wait_send()
            x_rdma(j).wait_send()
        for k in range(2 * YC):
            z_rdma(k, zoff).wait_send()

        s_rem = jnp.sum(
            recv_buf[...].astype(jnp.float32), axis=1, keepdims=True
        )
        inv = 1.0 / (s_loc + s_rem)
        loc_off = my_y * HALF
        rem_off = (1 - my_y) * HALF

        copies = []
        for t in range(NT):
            cs = pl.ds(t * TCHUNK, TCHUNK)
            send_buf[:, cs] = (
                send_buf[:, cs].astype(jnp.float32) * inv
            ).astype(jnp.bfloat16)
            c = pltpu.make_async_copy(
                send_buf.at[:, cs],
                out_hbm.at[:, pl.ds(loc_off + t * TCHUNK, TCHUNK)],
                out_sems.at[t],
            )
            c.start()
            copies.append(c)
        for t in range(NT):
            cs = pl.ds(t * TCHUNK, TCHUNK)
            recv_buf[:, cs] = (
                recv_buf[:, cs].astype(jnp.float32) * inv
            ).astype(jnp.bfloat16)
            c = pltpu.make_async_copy(
                recv_buf.at[:, cs],
                out_hbm.at[:, pl.ds(rem_off + t * TCHUNK, TCHUNK)],
                out_sems.at[NT + t],
            )
            c.start()
            copies.append(c)

        for c in copies:
            c.wait()

    return pl.pallas_call(
        body,
        out_shape=jax.ShapeDtypeStruct((ROWS, 2 * HALF), jnp.bfloat16),
        in_specs=[
            pl.BlockSpec(memory_space=pltpu.MemorySpace.HBM),
            pl.BlockSpec(memory_space=pltpu.MemorySpace.HBM),
        ],
        out_specs=pl.BlockSpec(memory_space=pltpu.MemorySpace.HBM),
        scratch_shapes=[
            pltpu.VMEM((ROWS, K), jnp.float32),
            pltpu.VMEM((2, K, Q), jnp.float32),
            pltpu.VMEM((ROWS, HALF), jnp.bfloat16),
            pltpu.VMEM((ROWS, HALF), jnp.bfloat16),
            pltpu.SemaphoreType.DMA,
            pltpu.SemaphoreType.DMA((2,)),
            pltpu.SemaphoreType.DMA((2 * NT,)),
            pltpu.SemaphoreType.DMA((YC,)),
            pltpu.SemaphoreType.DMA((YC,)),
            pltpu.SemaphoreType.DMA((YC,)),
            pltpu.SemaphoreType.DMA((YC,)),
            pltpu.SemaphoreType.DMA((2 * YC,)),
            pltpu.SemaphoreType.DMA((2 * YC,)),
        ],
        compiler_params=pltpu.CompilerParams(collective_id=0),
    )(
        pltpu.with_memory_space_constraint(x, pltpu.MemorySpace.HBM),
        pltpu.with_memory_space_constraint(W, pltpu.MemorySpace.HBM),
    )


# device time: 24210 ns/iter; 1.5603x vs baseline; 1.0830x over previous
import jax
import jax.numpy as jnp
from jax import lax
from jax.experimental import pallas as pl
from jax.experimental.pallas import tpu as pltpu

ROWS = 256
K = 512
HALF = 4096
Q = HALF // 4
YC = 4
YCW = Q // YC
NREST = 3
NT = 4
TCHUNK = HALF // NT


def kernel(x, W):
    def body(
        x_hbm,
        w_hbm,
        out_hbm,
        x_vmem,
        w_vmem,
        send_buf,
        recv_buf,
        x_sem,
        w_sems,
        out_sems,
        y_send, y_recv,
        xf_send, xf_recv,
        z_send, z_recv,
    ):
        my_x = lax.axis_index("x")
        my_y = lax.axis_index("y")
        my_z = lax.axis_index("z")
        p_y = (my_x, 1 - my_y, my_z)
        p_x = (1 - my_x, my_y, my_z)
        p_z = (my_x, my_y, 1 - my_z)

        qoff = (2 * my_z + my_x) * Q
        qxoff = (2 * my_z + (1 - my_x)) * Q
        qzoff = (2 * (1 - my_z) + my_x) * Q
        qdoff = (2 * (1 - my_z) + (1 - my_x)) * Q

        x_copy = pltpu.make_async_copy(x_hbm, x_vmem, x_sem)
        x_copy.start()

        sched = [(qoff + j * YCW, YCW) for j in range(YC)]
        sched += [(lax.rem(qoff + Q + r * Q, HALF), Q) for r in range(NREST)]

        def w_copy(i):
            col, width = sched[i]
            return pltpu.make_async_copy(
                w_hbm.at[:, pl.ds(col, width)],
                w_vmem.at[i % 2, :, :width],
                w_sems.at[i % 2],
            )

        w_copy(0).start()

        barrier_sem = pltpu.get_barrier_semaphore()
        for nbr in (p_y, p_x, p_z):
            pl.semaphore_signal(
                barrier_sem, inc=1, device_id=nbr,
                device_id_type=pl.DeviceIdType.MESH,
            )
        pl.semaphore_wait(barrier_sem, 3)

        x_copy.wait()
        xl = x_vmem[...].astype(jnp.bfloat16)

        def rdma(src_off, dst_off, width, send_sems, recv_sems, idx, dev):
            return pltpu.make_async_remote_copy(
                src_ref=recv_buf.at[:, pl.ds(src_off, width)],
                dst_ref=recv_buf.at[:, pl.ds(dst_off, width)],
                send_sem=send_sems.at[idx],
                recv_sem=recv_sems.at[idx],
                device_id=dev,
                device_id_type=pl.DeviceIdType.MESH,
            )

        def y_rdma(j):
            cs = pl.ds(qoff + j * YCW, YCW)
            return pltpu.make_async_remote_copy(
                src_ref=send_buf.at[:, cs],
                dst_ref=recv_buf.at[:, cs],
                send_sem=y_send.at[j],
                recv_sem=y_recv.at[j],
                device_id=p_y,
                device_id_type=pl.DeviceIdType.MESH,
            )

        def x_fwd(j):
            o = qoff + j * YCW
            return rdma(o, o, YCW, xf_send, xf_recv, j, p_x)

        def x_relay(j):
            o = qzoff + j * YCW
            return rdma(o, o, YCW, xf_send, xf_recv, YC + j, p_x)

        def z_fwd(j):
            o = qoff + j * YCW
            return rdma(o, o, YCW, z_send, z_recv, j, p_z)

        def z_relay(j):
            o = qxoff + j * YCW
            return rdma(o, o, YCW, z_send, z_recv, YC + (j - 2), p_z)

        s_loc = jnp.zeros((ROWS, 1), jnp.float32)
        for i in range(len(sched)):
            col, width = sched[i]
            w_copy(i).wait()
            if i + 1 < len(sched):
                w_copy(i + 1).start()
            wk = w_vmem[i % 2, :, :width].astype(jnp.bfloat16)
            ek = jnp.exp(jnp.dot(xl, wk, preferred_element_type=jnp.float32))
            send_buf[:, pl.ds(col, width)] = ek.astype(jnp.bfloat16)
            if i < YC:
                y_rdma(i).start()
            s_loc = s_loc + jnp.sum(ek, axis=1, keepdims=True)
            if YC <= i:
                j = i - YC
                y_rdma(j).wait_recv()
                x_fwd(j).start()
                z_fwd(j).start()

        for j in range(NREST, YC):
            y_rdma(j).wait_recv()
            x_fwd(j).start()
            z_fwd(j).start()

        for j in (0, 1):
            z_fwd(j).wait_recv()
            x_relay(j).start()
        for j in (2, 3):
            x_fwd(j).wait_recv()
            z_relay(j).start()

        for j in (0, 1):
            x_fwd(j).wait_recv()
        for j in (2, 3):
            z_fwd(j).wait_recv()
        for j in (0, 1):
            x_relay(j).wait_recv()
            z_relay(j + 2).wait_recv()

        for j in range(YC):
            y_rdma(j).wait_send()
            x_fwd(j).wait_send()
            z_fwd(j).wait_send()
        for j in (0, 1):
            x_relay(j).wait_send()
            z_relay(j + 2).wait_send()

        s_rem = jnp.sum(
            recv_buf[...].astype(jnp.float32), axis=1, keepdims=True
        )
        inv = 1.0 / (s_loc + s_rem)
        loc_off = my_y * HALF
        rem_off = (1 - my_y) * HALF

        copies = []
        for t in range(NT):
            cs = pl.ds(t * TCHUNK, TCHUNK)
            send_buf[:, cs] = (
                send_buf[:, cs].astype(jnp.float32) * inv
            ).astype(jnp.bfloat16)
            c = pltpu.make_async_copy(
                send_buf.at[:, cs],
                out_hbm.at[:, pl.ds(loc_off + t * TCHUNK, TCHUNK)],
                out_sems.at[t],
            )
            c.start()
            copies.append(c)
        for t in range(NT):
            cs = pl.ds(t * TCHUNK, TCHUNK)
            recv_buf[:, cs] = (
                recv_buf[:, cs].astype(jnp.float32) * inv
            ).astype(jnp.bfloat16)
            c = pltpu.make_async_copy(
                recv_buf.at[:, cs],
                out_hbm.at[:, pl.ds(rem_off + t * TCHUNK, TCHUNK)],
                out_sems.at[NT + t],
            )
            c.start()
            copies.append(c)
        for c in copies:
            c.wait()

    return pl.pallas_call(
        body,
        out_shape=jax.ShapeDtypeStruct((ROWS, 2 * HALF), jnp.bfloat16),
        in_specs=[
            pl.BlockSpec(memory_space=pltpu.MemorySpace.HBM),
            pl.BlockSpec(memory_space=pltpu.MemorySpace.HBM),
        ],
        out_specs=pl.BlockSpec(memory_space=pltpu.MemorySpace.HBM),
        scratch_shapes=[
            pltpu.VMEM((ROWS, K), jnp.float32),
            pltpu.VMEM((2, K, Q), jnp.float32),
            pltpu.VMEM((ROWS, HALF), jnp.bfloat16),
            pltpu.VMEM((ROWS, HALF), jnp.bfloat16),
            pltpu.SemaphoreType.DMA,
            pltpu.SemaphoreType.DMA((2,)),
            pltpu.SemaphoreType.DMA((2 * NT,)),
            pltpu.SemaphoreType.DMA((YC,)),
            pltpu.SemaphoreType.DMA((YC,)),
            pltpu.SemaphoreType.DMA((YC + 2,)),
            pltpu.SemaphoreType.DMA((YC + 2,)),
            pltpu.SemaphoreType.DMA((YC + 2,)),
            pltpu.SemaphoreType.DMA((YC + 2,)),
        ],
        compiler_params=pltpu.CompilerParams(collective_id=0),
    )(
        pltpu.with_memory_space_constraint(x, pltpu.MemorySpace.HBM),
        pltpu.with_memory_space_constraint(W, pltpu.MemorySpace.HBM),
    )


# device time: 23281 ns/iter; 1.6225x vs baseline; 1.0399x over previous
import jax
import jax.numpy as jnp
from jax import lax
from jax.experimental import pallas as pl
from jax.experimental.pallas import tpu as pltpu

ROWS = 256
K = 512
HALF = 4096
Q = HALF // 4
YC = 4
YCW = Q // YC
NREST = 3


def kernel(x, W):
    def body(
        x_hbm,
        w_hbm,
        out_hbm,
        x_vmem,
        w_vmem,
        send_buf,
        recv_buf,
        stats_out,
        stats_in,
        x_sem,
        w_sems,
        out_sems,
        stats_sems,
        y_send, y_recv,
        xf_send, xf_recv,
        z_send, z_recv,
    ):
        my_x = lax.axis_index("x")
        my_y = lax.axis_index("y")
        my_z = lax.axis_index("z")
        p_y = (my_x, 1 - my_y, my_z)
        p_x = (1 - my_x, my_y, my_z)
        p_z = (my_x, my_y, 1 - my_z)

        qoff = (2 * my_z + my_x) * Q
        qxoff = (2 * my_z + (1 - my_x)) * Q
        qzoff = (2 * (1 - my_z) + my_x) * Q
        qdoff = (2 * (1 - my_z) + (1 - my_x)) * Q

        x_copy = pltpu.make_async_copy(x_hbm, x_vmem, x_sem)
        x_copy.start()

        sched = [(qoff + j * YCW, YCW) for j in range(YC)]
        sched += [(lax.rem(qoff + Q + r * Q, HALF), Q) for r in range(NREST)]

        def w_copy(i):
            col, width = sched[i]
            return pltpu.make_async_copy(
                w_hbm.at[:, pl.ds(col, width)],
                w_vmem.at[i % 2, :, :width],
                w_sems.at[i % 2],
            )

        w_copy(0).start()

        barrier_sem = pltpu.get_barrier_semaphore()
        for nbr in (p_y, p_x, p_z):
            pl.semaphore_signal(
                barrier_sem, inc=1, device_id=nbr,
                device_id_type=pl.DeviceIdType.MESH,
            )
        pl.semaphore_wait(barrier_sem, 3)

        x_copy.wait()
        xl = x_vmem[...].astype(jnp.bfloat16)

        def rdma(src_off, dst_off, width, send_sems, recv_sems, idx, dev):
            return pltpu.make_async_remote_copy(
                src_ref=recv_buf.at[:, pl.ds(src_off, width)],
                dst_ref=recv_buf.at[:, pl.ds(dst_off, width)],
                send_sem=send_sems.at[idx],
                recv_sem=recv_sems.at[idx],
                device_id=dev,
                device_id_type=pl.DeviceIdType.MESH,
            )

        def y_rdma(j):
            cs = pl.ds(qoff + j * YCW, YCW)
            return pltpu.make_async_remote_copy(
                src_ref=send_buf.at[:, cs],
                dst_ref=recv_buf.at[:, cs],
                send_sem=y_send.at[j],
                recv_sem=y_recv.at[j],
                device_id=p_y,
                device_id_type=pl.DeviceIdType.MESH,
            )

        stats_rdma = pltpu.make_async_remote_copy(
            src_ref=stats_out,
            dst_ref=stats_in,
            send_sem=stats_sems.at[0],
            recv_sem=stats_sems.at[1],
            device_id=p_y,
            device_id_type=pl.DeviceIdType.MESH,
        )

        def x_fwd(j):
            o = qoff + j * YCW
            return rdma(o, o, YCW, xf_send, xf_recv, j, p_x)

        def x_relay(j):
            o = qzoff + j * YCW
            return rdma(o, o, YCW, xf_send, xf_recv, YC + j, p_x)

        def z_fwd(j):
            o = qoff + j * YCW
            return rdma(o, o, YCW, z_send, z_recv, j, p_z)

        def z_relay(j):
            o = qxoff + j * YCW
            return rdma(o, o, YCW, z_send, z_recv, YC + (j - 2), p_z)

        s_loc = jnp.zeros((ROWS, 1), jnp.float32)
        for i in range(len(sched)):
            col, width = sched[i]
            w_copy(i).wait()
            if i + 1 < len(sched):
                w_copy(i + 1).start()
            wk = w_vmem[i % 2, :, :width].astype(jnp.bfloat16)
            ek = jnp.exp(jnp.dot(xl, wk, preferred_element_type=jnp.float32))
            send_buf[:, pl.ds(col, width)] = ek.astype(jnp.bfloat16)
            if i < YC:
                y_rdma(i).start()
            s_loc = s_loc + jnp.sum(ek, axis=1, keepdims=True)
            if YC <= i:
                j = i - YC
                y_rdma(j).wait_recv()
                x_fwd(j).start()
                z_fwd(j).start()

        stats_out[...] = jnp.broadcast_to(
            s_loc.astype(jnp.bfloat16), (ROWS, 128)
        )
        stats_rdma.start()

        for j in range(NREST, YC):
            y_rdma(j).wait_recv()
            x_fwd(j).start()
            z_fwd(j).start()

        for j in (0, 1):
            z_fwd(j).wait_recv()
            x_relay(j).start()
        for j in (2, 3):
            x_fwd(j).wait_recv()
            z_relay(j).start()

        stats_rdma.wait_recv()
        inv = 1.0 / (s_loc + stats_in[:, 0:1].astype(jnp.float32))
        loc_off = my_y * HALF
        rem_off = (1 - my_y) * HALF

        copies = []

        def scale_store(buf, off, width, half_off, sem_idx):
            cs = pl.ds(off, width)
            buf[:, cs] = (buf[:, cs].astype(jnp.float32) * inv).astype(
                jnp.bfloat16
            )
            c = pltpu.make_async_copy(
                buf.at[:, cs],
                out_hbm.at[:, pl.ds(half_off + off, width)],
                out_sems.at[sem_idx],
            )
            c.start()
            copies.append(c)

        for j in range(YC):
            y_rdma(j).wait_send()
        stats_rdma.wait_send()
        for t in range(4):
            scale_store(send_buf, t * Q, Q, loc_off, t)

        for j in (0, 1):
            x_fwd(j).wait_recv()
            scale_store(recv_buf, qxoff + j * YCW, YCW, rem_off, 4 + j)
        for j in (2, 3):
            z_fwd(j).wait_recv()
            scale_store(recv_buf, qzoff + j * YCW, YCW, rem_off, 6 + (j - 2))
        for j in (0, 1):
            x_relay(j).wait_recv()
            scale_store(recv_buf, qdoff + j * YCW, YCW, rem_off, 8 + j)
            z_relay(j + 2).wait_recv()
            scale_store(recv_buf, qdoff + (2 + j) * YCW, YCW, rem_off, 10 + j)

        for j in range(YC):
            x_fwd(j).wait_send()
            z_fwd(j).wait_send()
        scale_store(recv_buf, qoff, Q, rem_off, 12)
        for j in (0, 1):
            x_relay(j).wait_send()
            z_relay(j + 2).wait_send()
        for j in (0, 1):
            scale_store(recv_buf, qzoff + j * YCW, YCW, rem_off, 13 + j)
            scale_store(recv_buf, qxoff + (2 + j) * YCW, YCW, rem_off, 15 + j)

        for c in copies:
            c.wait()

    return pl.pallas_call(
        body,
        out_shape=jax.ShapeDtypeStruct((ROWS, 2 * HALF), jnp.bfloat16),
        in_specs=[
            pl.BlockSpec(memory_space=pltpu.MemorySpace.HBM),
            pl.BlockSpec(memory_space=pltpu.MemorySpace.HBM),
        ],
        out_specs=pl.BlockSpec(memory_space=pltpu.MemorySpace.HBM),
        scratch_shapes=[
            pltpu.VMEM((ROWS, K), jnp.float32),
            pltpu.VMEM((2, K, Q), jnp.float32),
            pltpu.VMEM((ROWS, HALF), jnp.bfloat16),
            pltpu.VMEM((ROWS, HALF), jnp.bfloat16),
            pltpu.VMEM((ROWS, 128), jnp.bfloat16),
            pltpu.VMEM((ROWS, 128), jnp.bfloat16),
            pltpu.SemaphoreType.DMA,
            pltpu.SemaphoreType.DMA((2,)),
            pltpu.SemaphoreType.DMA((17,)),
            pltpu.SemaphoreType.DMA((2,)),
            pltpu.SemaphoreType.DMA((YC,)),
            pltpu.SemaphoreType.DMA((YC,)),
            pltpu.SemaphoreType.DMA((YC + 2,)),
            pltpu.SemaphoreType.DMA((YC + 2,)),
            pltpu.SemaphoreType.DMA((YC + 2,)),
            pltpu.SemaphoreType.DMA((YC + 2,)),
        ],
        compiler_params=pltpu.CompilerParams(collective_id=0),
    )(
        pltpu.with_memory_space_constraint(x, pltpu.MemorySpace.HBM),
        pltpu.with_memory_space_constraint(W, pltpu.MemorySpace.HBM),
    )


# device time: 22486 ns/iter; 1.6799x vs baseline; 1.0354x over previous
import jax
import jax.numpy as jnp
from jax import lax
from jax.experimental import pallas as pl
from jax.experimental.pallas import tpu as pltpu

ROWS = 256
K = 512
HALF = 4096
Q = HALF // 4
YC = 4
YCW = Q // YC
NREST = 3


def kernel(x, W):
    def body(
        x_hbm,
        w_hbm,
        out_hbm,
        x_vmem,
        w_vmem,
        send_buf,
        recv_buf,
        stats_out,
        stats_in,
        x_sem,
        w_sems,
        out_sems,
        stats_sems,
        y_send, y_recv,
        xf_send, xf_recv,
        z_send, z_recv,
    ):
        my_x = lax.axis_index("x")
        my_y = lax.axis_index("y")
        my_z = lax.axis_index("z")
        p_y = (my_x, 1 - my_y, my_z)
        p_x = (1 - my_x, my_y, my_z)
        p_z = (my_x, my_y, 1 - my_z)

        qoff = (2 * my_z + my_x) * Q
        qxoff = (2 * my_z + (1 - my_x)) * Q
        qzoff = (2 * (1 - my_z) + my_x) * Q
        qdoff = (2 * (1 - my_z) + (1 - my_x)) * Q

        x_copy = pltpu.make_async_copy(x_hbm, x_vmem, x_sem)
        x_copy.start()

        sched = [(qoff + j * YCW, YCW) for j in range(YC)]
        sched += [(qdoff, Q), (qxoff, Q), (qzoff, Q)]

        def w_copy(i):
            col, width = sched[i]
            return pltpu.make_async_copy(
                w_hbm.at[:, pl.ds(col, width)],
                w_vmem.at[i % 2, :, :width],
                w_sems.at[i % 2],
            )

        w_copy(0).start()

        barrier_sem = pltpu.get_barrier_semaphore()
        for nbr in (p_y, p_x, p_z):
            pl.semaphore_signal(
                barrier_sem, inc=1, device_id=nbr,
                device_id_type=pl.DeviceIdType.MESH,
            )
        pl.semaphore_wait(barrier_sem, 3)

        x_copy.wait()
        xl = x_vmem[...].astype(jnp.bfloat16)

        def rdma(src_off, dst_off, width, send_sems, recv_sems, idx, dev):
            return pltpu.make_async_remote_copy(
                src_ref=recv_buf.at[:, pl.ds(src_off, width)],
                dst_ref=recv_buf.at[:, pl.ds(dst_off, width)],
                send_sem=send_sems.at[idx],
                recv_sem=recv_sems.at[idx],
                device_id=dev,
                device_id_type=pl.DeviceIdType.MESH,
            )

        def y_rdma(j):
            cs = pl.ds(qoff + j * YCW, YCW)
            return pltpu.make_async_remote_copy(
                src_ref=send_buf.at[:, cs],
                dst_ref=recv_buf.at[:, cs],
                send_sem=y_send.at[j],
                recv_sem=y_recv.at[j],
                device_id=p_y,
                device_id_type=pl.DeviceIdType.MESH,
            )

        def y_extra(j):
            cs = pl.ds(qdoff + (2 + j) * YCW, YCW)
            return pltpu.make_async_remote_copy(
                src_ref=send_buf.at[:, cs],
                dst_ref=recv_buf.at[:, cs],
                send_sem=y_send.at[YC + j],
                recv_sem=y_recv.at[YC + j],
                device_id=p_y,
                device_id_type=pl.DeviceIdType.MESH,
            )

        stats_rdma = pltpu.make_async_remote_copy(
            src_ref=stats_out,
            dst_ref=stats_in,
            send_sem=stats_sems.at[0],
            recv_sem=stats_sems.at[1],
            device_id=p_y,
            device_id_type=pl.DeviceIdType.MESH,
        )

        def x_fwd(j):
            o = qoff + j * YCW
            return rdma(o, o, YCW, xf_send, xf_recv, j, p_x)

        def x_relay(j):
            o = qzoff + j * YCW
            return rdma(o, o, YCW, xf_send, xf_recv, YC + j, p_x)

        def z_fwd(j):
            o = qoff + j * YCW
            return rdma(o, o, YCW, z_send, z_recv, j, p_z)

        s_loc = jnp.zeros((ROWS, 1), jnp.float32)
        for i in range(len(sched)):
            col, width = sched[i]
            w_copy(i).wait()
            if i + 1 < len(sched):
                w_copy(i + 1).start()
            wk = w_vmem[i % 2, :, :width].astype(jnp.bfloat16)
            ek = jnp.exp(jnp.dot(xl, wk, preferred_element_type=jnp.float32))
            send_buf[:, pl.ds(col, width)] = ek.astype(jnp.bfloat16)
            if i < YC:
                y_rdma(i).start()
            s_loc = s_loc + jnp.sum(ek, axis=1, keepdims=True)
            if i == YC:
                y_extra(0).start()
                y_extra(1).start()
            if YC <= i:
                j = i - YC
                y_rdma(j).wait_recv()
                x_fwd(j).start()
                z_fwd(j).start()

        stats_out[...] = jnp.broadcast_to(
            s_loc.astype(jnp.bfloat16), (ROWS, 128)
        )
        stats_rdma.start()

        for j in range(NREST, YC):
            y_rdma(j).wait_recv()
            x_fwd(j).start()
            z_fwd(j).start()

        for j in (0, 1):
            z_fwd(j).wait_recv()
            x_relay(j).start()

        stats_rdma.wait_recv()
        inv = 1.0 / (s_loc + stats_in[:, 0:1].astype(jnp.float32))
        loc_off = my_y * HALF
        rem_off = (1 - my_y) * HALF

        copies = []

        def scale_store(buf, off, width, half_off, sem_idx):
            cs = pl.ds(off, width)
            buf[:, cs] = (buf[:, cs].astype(jnp.float32) * inv).astype(
                jnp.bfloat16
            )
            c = pltpu.make_async_copy(
                buf.at[:, cs],
                out_hbm.at[:, pl.ds(half_off + off, width)],
                out_sems.at[sem_idx],
            )
            c.start()
            copies.append(c)

        for j in range(YC):
            y_rdma(j).wait_send()
        y_extra(0).wait_send()
        y_extra(1).wait_send()
        stats_rdma.wait_send()
        for t in range(4):
            scale_store(send_buf, t * Q, Q, loc_off, t)

        for j in range(YC):
            x_fwd(j).wait_recv()
            scale_store(recv_buf, qxoff + j * YCW, YCW, rem_off, 4 + j)
        for j in (2, 3):
            z_fwd(j).wait_recv()
            scale_store(recv_buf, qzoff + j * YCW, YCW, rem_off, 8 + (j - 2))
        for j in (0, 1):
            x_relay(j).wait_recv()
            scale_store(recv_buf, qdoff + j * YCW, YCW, rem_off, 10 + j)
            y_extra(j).wait_recv()
            scale_store(recv_buf, qdoff + (2 + j) * YCW, YCW, rem_off, 12 + j)

        for j in range(YC):
            x_fwd(j).wait_send()
            z_fwd(j).wait_send()
        scale_store(recv_buf, qoff, Q, rem_off, 14)
        for j in (0, 1):
            x_relay(j).wait_send()
        for j in (0, 1):
            scale_store(recv_buf, qzoff + j * YCW, YCW, rem_off, 15 + j)

        for c in copies:
            c.wait()

    return pl.pallas_call(
        body,
        out_shape=jax.ShapeDtypeStruct((ROWS, 2 * HALF), jnp.bfloat16),
        in_specs=[
            pl.BlockSpec(memory_space=pltpu.MemorySpace.HBM),
            pl.BlockSpec(memory_space=pltpu.MemorySpace.HBM),
        ],
        out_specs=pl.BlockSpec(memory_space=pltpu.MemorySpace.HBM),
        scratch_shapes=[
            pltpu.VMEM((ROWS, K), jnp.float32),
            pltpu.VMEM((2, K, Q), jnp.float32),
            pltpu.VMEM((ROWS, HALF), jnp.bfloat16),
            pltpu.VMEM((ROWS, HALF), jnp.bfloat16),
            pltpu.VMEM((ROWS, 128), jnp.bfloat16),
            pltpu.VMEM((ROWS, 128), jnp.bfloat16),
            pltpu.SemaphoreType.DMA,
            pltpu.SemaphoreType.DMA((2,)),
            pltpu.SemaphoreType.DMA((17,)),
            pltpu.SemaphoreType.DMA((2,)),
            pltpu.SemaphoreType.DMA((YC + 2,)),
            pltpu.SemaphoreType.DMA((YC + 2,)),
            pltpu.SemaphoreType.DMA((YC + 2,)),
            pltpu.SemaphoreType.DMA((YC + 2,)),
            pltpu.SemaphoreType.DMA((YC,)),
            pltpu.SemaphoreType.DMA((YC,)),
        ],
        compiler_params=pltpu.CompilerParams(collective_id=0),
    )(
        pltpu.with_memory_space_constraint(x, pltpu.MemorySpace.HBM),
        pltpu.with_memory_space_constraint(W, pltpu.MemorySpace.HBM),
    )
